# baseline (speedup 1.0000x reference)
import functools

import jax
import jax.numpy as jnp
from jax.experimental import pallas as pl
from jax.experimental.pallas import tpu as pltpu

_SLOTS = 64


def _mlp_body(x_ref, g1_ref, b1_ref, g2_ref, b2_ref, o_ref):
    h = jnp.dot(x_ref[...], g1_ref[...], preferred_element_type=jnp.float32)
    h = jnp.tanh(h + b1_ref[...])
    y = jnp.dot(h, g2_ref[...], preferred_element_type=jnp.float32)
    o_ref[...] = y + b2_ref[...]


_RESIDENT = pl.BlockSpec(memory_space=pltpu.MemorySpace.VMEM)


@functools.partial(jax.jit, static_argnames=("block_rows",))
def _forward(x, w1, b1, w2, b2, block_rows=512):
    B, S = x.shape
    H = w1.shape[0]
    A = w2.shape[0]

    w1 = w1.astype(jnp.float32)
    b1 = b1.astype(jnp.float32).reshape(-1)
    w2 = w2.astype(jnp.float32)
    b2 = b2.astype(jnp.float32).reshape(-1)

    eye = jnp.eye(_SLOTS, dtype=jnp.float32)
    g1 = jnp.kron(eye, w1.T)
    g2 = jnp.kron(eye, w2.T)
    b1r = jnp.tile(b1, _SLOTS)[None, :]
    b2r = jnp.tile(b2, _SLOTS)[None, :]

    rows = -(-B // _SLOTS)
    grid_rows = -(-rows // block_rows) * block_rows
    b_pad = grid_rows * _SLOTS
    if b_pad != B:
        x = jnp.pad(x, ((0, b_pad - B), (0, 0)))
    xr = x.reshape(grid_rows, _SLOTS * S)

    out = pl.pallas_call(
        _mlp_body,
        out_shape=jax.ShapeDtypeStruct((grid_rows, _SLOTS * A), jnp.float32),
        grid=(grid_rows // block_rows,),
        in_specs=[
            pl.BlockSpec((block_rows, _SLOTS * S), lambda i: (i, 0)),
            _RESIDENT, _RESIDENT, _RESIDENT, _RESIDENT,
        ],
        out_specs=pl.BlockSpec((block_rows, _SLOTS * A), lambda i: (i, 0)),
        compiler_params=pltpu.CompilerParams(
            dimension_semantics=("parallel",)),
    )(xr, g1, b1r, g2, b2r)

    return out.reshape(b_pad, A)[:B]


def kernel(x, w1, b1, w2, b2):
    return _forward(x, w1, b1, w2, b2)


# speedup vs baseline: 38.8900x; 38.8900x over previous
import functools

import jax
import jax.numpy as jnp
from jax.experimental import pallas as pl
from jax.experimental.pallas import tpu as pltpu

_H_PAD = 56


def _mlp_body(xt_ref, w1_ref, b1_ref, w2_ref, b2_ref, o_ref):
    ht = jnp.dot(w1_ref[...], xt_ref[...], preferred_element_type=jnp.float32)
    ht = jnp.tanh(ht + b1_ref[...])
    o_ref[...] = (
        jnp.dot(w2_ref[...], ht, preferred_element_type=jnp.float32)
        + b2_ref[...])


_RESIDENT = pl.BlockSpec(memory_space=pltpu.MemorySpace.VMEM)


@functools.partial(jax.jit, static_argnames=("block_b",))
def _forward(x, w1, b1, w2, b2, block_b=32768):
    B, S = x.shape
    H = w1.shape[0]
    A = w2.shape[0]

    w1 = w1.astype(jnp.float32)
    b1 = b1.astype(jnp.float32).reshape(-1)
    w2 = w2.astype(jnp.float32)
    b2 = b2.astype(jnp.float32).reshape(-1)

    w1p = jnp.zeros((_H_PAD, S), jnp.float32).at[:H, :].set(w1)
    b1p = jnp.zeros((_H_PAD, 1), jnp.float32).at[:H, 0].set(b1)
    w2p = jnp.zeros((A, _H_PAD), jnp.float32).at[:, :H].set(w2)
    b2p = b2[:, None]

    b_pad = -(-B // block_b) * block_b
    xt = jnp.zeros((S, b_pad), jnp.float32).at[:, :B].set(x.T)

    yt = pl.pallas_call(
        _mlp_body,
        out_shape=jax.ShapeDtypeStruct((A, b_pad), jnp.float32),
        grid=(b_pad // block_b,),
        in_specs=[
            pl.BlockSpec((S, block_b), lambda i: (0, i)),
            _RESIDENT, _RESIDENT, _RESIDENT, _RESIDENT,
        ],
        out_specs=pl.BlockSpec((A, block_b), lambda i: (0, i)),
        compiler_params=pltpu.CompilerParams(
            dimension_semantics=("parallel",)),
    )(xt, w1p, b1p, w2p, b2p)

    return yt[:, :B].T


def kernel(x, w1, b1, w2, b2):
    return _forward(x, w1, b1, w2, b2)


# speedup vs baseline: 41.0705x; 1.0561x over previous
import functools

import jax
import jax.numpy as jnp
from jax.experimental import pallas as pl
from jax.experimental.pallas import tpu as pltpu

_H_PAD = 56


def _mlp_body(xt_ref, w1_ref, b1_ref, w2_ref, b2_ref, o_ref):
    ht = jnp.dot(w1_ref[...], xt_ref[...], preferred_element_type=jnp.float32)
    ht = jnp.tanh(ht + b1_ref[...])
    o_ref[...] = (
        jnp.dot(w2_ref[...], ht, preferred_element_type=jnp.float32)
        + b2_ref[...])


_RESIDENT = pl.BlockSpec(memory_space=pltpu.MemorySpace.VMEM)


@functools.partial(jax.jit, static_argnames=("block_b",))
def _forward(x, w1, b1, w2, b2, block_b=65536):
    B, S = x.shape
    H = w1.shape[0]
    A = w2.shape[0]

    w1 = w1.astype(jnp.float32)
    b1 = b1.astype(jnp.float32).reshape(-1)
    w2 = w2.astype(jnp.float32)
    b2 = b2.astype(jnp.float32).reshape(-1)

    w1p = jnp.zeros((_H_PAD, S), jnp.float32).at[:H, :].set(w1)
    b1p = jnp.zeros((_H_PAD, 1), jnp.float32).at[:H, 0].set(b1)
    w2p = jnp.zeros((A, _H_PAD), jnp.float32).at[:, :H].set(w2)
    b2p = b2[:, None]

    b_pad = -(-B // block_b) * block_b
    xt = jnp.zeros((S, b_pad), jnp.float32).at[:, :B].set(x.T)

    yt = pl.pallas_call(
        _mlp_body,
        out_shape=jax.ShapeDtypeStruct((A, b_pad), jnp.float32),
        grid=(b_pad // block_b,),
        in_specs=[
            pl.BlockSpec((S, block_b), lambda i: (0, i)),
            _RESIDENT, _RESIDENT, _RESIDENT, _RESIDENT,
        ],
        out_specs=pl.BlockSpec((A, block_b), lambda i: (0, i)),
        compiler_params=pltpu.CompilerParams(
            dimension_semantics=("parallel",)),
    )(xt, w1p, b1p, w2p, b2p)

    return yt[:, :B].T


def kernel(x, w1, b1, w2, b2):
    return _forward(x, w1, b1, w2, b2)
